# Initial kernel scaffold; baseline (speedup 1.0000x reference)
import jax
import jax.numpy as jnp
from jax.experimental import pallas as pl


def kernel(x, token_embd, pos_embd):
    raise NotImplementedError("write your pallas kernel here")


# speedup vs baseline: 1.0831x; 1.0831x over previous
import functools

import jax
import jax.numpy as jnp
from jax import lax
from jax.experimental import pallas as pl
from jax.experimental.pallas import tpu as pltpu
from jax.experimental.pallas import tpu_sc as plsc

B, S, D = 16, 2048, 128
L = 16

_info = plsc.get_sparse_core_info()
NC, NS = _info.num_cores, _info.num_subcores
NW = NC * NS
SW = S // NW


def _embed_body(x_hbm, tok_hbm, pos_hbm, out_hbm,
                idx_v, pos_v, buf0, buf1, sem0, sem1):
    wid = lax.axis_index("s") * NC + lax.axis_index("c")
    s0 = wid * SW

    for b in range(B):
        pltpu.sync_copy(x_hbm.at[pl.ds(b * S + s0, SW)], idx_v.at[b])
    pltpu.sync_copy(pos_hbm.at[pl.ds(s0, SW)], pos_v)

    bufs = (buf0, buf1)
    sems = (sem0, sem1)

    def start(b):
        return pltpu.async_copy(tok_hbm.at[idx_v.at[b]], bufs[b % 2], sems[b % 2])

    cp = start(0)
    for b in range(B):
        buf = bufs[b % 2]
        cur = cp
        if b + 1 < B:
            cp = start(b + 1)
        cur.wait()

        def add_row(r, carry, buf=buf):
            for c in range(D // L):
                sl = pl.ds(c * L, L)
                buf[r, sl] = buf[r, sl] + pos_v[r, sl]
            return carry

        lax.fori_loop(0, SW, add_row, 0)
        pltpu.sync_copy(buf, out_hbm.at[b, pl.ds(s0, SW)])


_embed = functools.partial(
    pl.kernel,
    out_type=jax.ShapeDtypeStruct((B, S, D), jnp.float32),
    mesh=plsc.VectorSubcoreMesh(core_axis_name="c", subcore_axis_name="s"),
    scratch_types=[
        pltpu.VMEM((B, SW), jnp.int32),
        pltpu.VMEM((SW, D), jnp.float32),
        pltpu.VMEM((SW, D), jnp.float32),
        pltpu.VMEM((SW, D), jnp.float32),
        pltpu.SemaphoreType.DMA,
        pltpu.SemaphoreType.DMA,
    ],
)(_embed_body)


def kernel(x, token_embd, pos_embd):
    return _embed(x.astype(jnp.int32).reshape(B * S), token_embd, pos_embd)
